# baseline (speedup 1.0000x reference)
import functools

import jax
import jax.numpy as jnp
from jax import lax
from jax.experimental import pallas as pl
from jax.experimental.pallas import tpu as pltpu
from jax.experimental.pallas import tpu_sc as plsc

TOPK = 8
NUM_EXPERTS = 64
BR = 1024
N_ROWS = 32768
NC, NS, L = 2, 16, 16
NW = NC * NS
RPW = (N_ROWS // 2) // NW


def _mm_block(x_ref, wt_ref, b_ref, o_ref):
    xb = x_ref[...]
    wt = wt_ref[...]
    logits = jax.lax.dot_general(
        xb, wt, dimension_numbers=(((1,), (0,)), ((), ())),
        preferred_element_type=jnp.float32,
    )
    o_ref[...] = logits + b_ref[...]


N_CHUNKS = 2
CHUNK_ROWS = N_ROWS // N_CHUNKS


def _gate_logits(x, wt, b2, chunk):
    n_rows, d = x.shape
    blk0 = chunk * (CHUNK_ROWS // BR)
    return pl.pallas_call(
        _mm_block,
        grid=(CHUNK_ROWS // BR,),
        in_specs=[
            pl.BlockSpec((BR, d), lambda i: (i + blk0, 0)),
            pl.BlockSpec((d, NUM_EXPERTS), lambda i: (0, 0)),
            pl.BlockSpec((1, NUM_EXPERTS), lambda i: (0, 0)),
        ],
        out_specs=pl.BlockSpec((BR, NUM_EXPERTS), lambda i: (i, 0)),
        out_shape=jax.ShapeDtypeStruct((CHUNK_ROWS, NUM_EXPERTS), jnp.float32),
    )(x, wt, b2)


def _halver(ak, av, bk, bv):
    take_a = ak >= bk
    mk = jnp.where(take_a, ak, bk)
    mv = jnp.where(take_a, av, bv)
    return mk, mv


CH = 256


def _topk_body(logits_hbm, w_hbm, i_hbm, slab, wq, iq):
    wid = lax.axis_index("s") * NC + lax.axis_index("c")
    base = wid * RPW

    iota = lax.iota(jnp.int32, L)
    lane_lt8 = iota < TOPK
    idx_c = [iota + L * c for c in range(4)]

    UNROLL = 4

    def one_row(r):
        ks = [slab[r, pl.ds(L * c, L)] for c in range(4)]
        s0 = plsc.sort_key_val(ks[0], idx_c[0], descending=True)
        s1 = plsc.sort_key_val(ks[1], idx_c[1], descending=False)
        s2 = plsc.sort_key_val(ks[2], idx_c[2], descending=True)
        s3 = plsc.sort_key_val(ks[3], idx_c[3], descending=False)
        c01 = _halver(s0[0], s0[1], s1[0], s1[1])
        c23 = _halver(s2[0], s2[1], s3[0], s3[1])
        m01 = plsc.sort_key_val(c01[0], c01[1], descending=True)
        m23 = plsc.sort_key_val(c23[0], c23[1], descending=False)
        cf = _halver(m01[0], m01[1], m23[0], m23[1])
        fk, fv = plsc.sort_key_val(cf[0], cf[1], descending=True)

        m = jnp.max(fk)
        e = jnp.exp(fk - m)
        e = jnp.where(lane_lt8, e, jnp.float32(0.0))
        s = jnp.sum(e)
        w = e / s

        row_vec = jnp.full((L,), r, jnp.int32)
        plsc.store_scatter(wq, [row_vec, iota], w, mask=lane_lt8)
        plsc.store_scatter(iq, [row_vec, iota], fv, mask=lane_lt8)

    for ch in range(RPW // CH):
        cbase = base + ch * CH
        pltpu.sync_copy(logits_hbm.at[pl.ds(cbase, CH)], slab)

        @plsc.parallel_loop(0, CH, step=1, unroll=UNROLL)
        def _(r):
            one_row(r)

        pltpu.sync_copy(wq, w_hbm.at[pl.ds(cbase, CH)])
        pltpu.sync_copy(iq, i_hbm.at[pl.ds(cbase, CH)])


_topk_sc = functools.partial(
    pl.kernel,
    out_type=(
        jax.ShapeDtypeStruct((N_ROWS // 2, TOPK), jnp.float32),
        jax.ShapeDtypeStruct((N_ROWS // 2, TOPK), jnp.int32),
    ),
    mesh=plsc.VectorSubcoreMesh(core_axis_name="c", subcore_axis_name="s"),
    scratch_types=[
        pltpu.VMEM((CH, NUM_EXPERTS), jnp.float32),
        pltpu.VMEM((CH, TOPK), jnp.float32),
        pltpu.VMEM((CH, TOPK), jnp.int32),
    ],
    compiler_params=pltpu.CompilerParams(needs_layout_passes=False),
)(_topk_body)


def kernel(x, W, b):
    wt = W.T
    b2 = b.reshape(1, NUM_EXPERTS)
    l0 = _gate_logits(x, wt, b2, 0)
    l1 = _gate_logits(x, wt, b2, 1)
    w0, i0 = _topk_sc(l0)
    w1, i1 = _topk_sc(l1)
    return (jnp.concatenate([w0, w1], axis=0),
            jnp.concatenate([i0, i1], axis=0))


# speedup vs baseline: 1.0050x; 1.0050x over previous
import functools

import jax
import jax.numpy as jnp
from jax import lax
from jax.experimental import pallas as pl
from jax.experimental.pallas import tpu as pltpu
from jax.experimental.pallas import tpu_sc as plsc

TOPK = 8
NUM_EXPERTS = 64
BR = 1024
N_ROWS = 32768
NC, NS, L = 2, 16, 16
NW = NC * NS
RPW = (N_ROWS // 2) // NW


def _mm_block(x_ref, wt_ref, b_ref, o_ref):
    xb = x_ref[...]
    wt = wt_ref[...]
    logits = jax.lax.dot_general(
        xb, wt, dimension_numbers=(((1,), (0,)), ((), ())),
        preferred_element_type=jnp.float32,
    )
    o_ref[...] = logits + b_ref[...]


N_CHUNKS = 2
CHUNK_ROWS = N_ROWS // N_CHUNKS


def _gate_logits(x, wt, b2, chunk):
    n_rows, d = x.shape
    blk0 = chunk * (CHUNK_ROWS // BR)
    return pl.pallas_call(
        _mm_block,
        grid=(CHUNK_ROWS // BR,),
        in_specs=[
            pl.BlockSpec((BR, d), lambda i: (i + blk0, 0)),
            pl.BlockSpec((d, NUM_EXPERTS), lambda i: (0, 0)),
            pl.BlockSpec((1, NUM_EXPERTS), lambda i: (0, 0)),
        ],
        out_specs=pl.BlockSpec((BR, NUM_EXPERTS), lambda i: (i, 0)),
        out_shape=jax.ShapeDtypeStruct((CHUNK_ROWS, NUM_EXPERTS), jnp.float32),
    )(x, wt, b2)


def _halver(ak, av, bk, bv):
    take_a = ak >= bk
    mk = jnp.where(take_a, ak, bk)
    mv = jnp.where(take_a, av, bv)
    return mk, mv


CH = 256


def _topk_body(logits_hbm, w_hbm, i_hbm, slab, wq, iq):
    wid = lax.axis_index("s") * NC + lax.axis_index("c")
    base = wid * RPW

    iota = lax.iota(jnp.int32, L)
    lane_lt8 = iota < TOPK
    idx_c = [iota + L * c for c in range(4)]
    zeros16 = jnp.zeros((L,), jnp.int32)
    fold4 = iota ^ 4
    fold2 = iota ^ 2
    fold1 = iota ^ 1

    UNROLL = 4

    def one_row(r):
        ks = [slab[r, pl.ds(L * c, L)] for c in range(4)]
        s0 = plsc.sort_key_val(ks[0], idx_c[0], descending=True)
        s1 = plsc.sort_key_val(ks[1], idx_c[1], descending=False)
        s2 = plsc.sort_key_val(ks[2], idx_c[2], descending=True)
        s3 = plsc.sort_key_val(ks[3], idx_c[3], descending=False)
        c01 = _halver(s0[0], s0[1], s1[0], s1[1])
        c23 = _halver(s2[0], s2[1], s3[0], s3[1])
        m01 = plsc.sort_key_val(c01[0], c01[1], descending=True)
        m23 = plsc.sort_key_val(c23[0], c23[1], descending=False)
        cf = _halver(m01[0], m01[1], m23[0], m23[1])
        fk, fv = plsc.sort_key_val(cf[0], cf[1], descending=True)

        m = fk.at[zeros16].get(mode="promise_in_bounds")
        e = jnp.exp(fk - m)
        e = jnp.where(lane_lt8, e, jnp.float32(0.0))
        s = e + e.at[fold4].get(mode="promise_in_bounds")
        s = s + s.at[fold2].get(mode="promise_in_bounds")
        s = s + s.at[fold1].get(mode="promise_in_bounds")
        w = e / s

        row_vec = jnp.full((L,), r, jnp.int32)
        plsc.store_scatter(wq, [row_vec, iota], w, mask=lane_lt8)
        plsc.store_scatter(iq, [row_vec, iota], fv, mask=lane_lt8)

    for ch in range(RPW // CH):
        cbase = base + ch * CH
        pltpu.sync_copy(logits_hbm.at[pl.ds(cbase, CH)], slab)

        @plsc.parallel_loop(0, CH, step=1, unroll=UNROLL)
        def _(r):
            one_row(r)

        pltpu.sync_copy(wq, w_hbm.at[pl.ds(cbase, CH)])
        pltpu.sync_copy(iq, i_hbm.at[pl.ds(cbase, CH)])


_topk_sc = functools.partial(
    pl.kernel,
    out_type=(
        jax.ShapeDtypeStruct((N_ROWS // 2, TOPK), jnp.float32),
        jax.ShapeDtypeStruct((N_ROWS // 2, TOPK), jnp.int32),
    ),
    mesh=plsc.VectorSubcoreMesh(core_axis_name="c", subcore_axis_name="s"),
    scratch_types=[
        pltpu.VMEM((CH, NUM_EXPERTS), jnp.float32),
        pltpu.VMEM((CH, TOPK), jnp.float32),
        pltpu.VMEM((CH, TOPK), jnp.int32),
    ],
    compiler_params=pltpu.CompilerParams(needs_layout_passes=False),
)(_topk_body)


def kernel(x, W, b):
    wt = W.T
    b2 = b.reshape(1, NUM_EXPERTS)
    l0 = _gate_logits(x, wt, b2, 0)
    l1 = _gate_logits(x, wt, b2, 1)
    w0, i0 = _topk_sc(l0)
    w1, i1 = _topk_sc(l1)
    return (jnp.concatenate([w0, w1], axis=0),
            jnp.concatenate([i0, i1], axis=0))
